# Initial kernel scaffold; baseline (speedup 1.0000x reference)
import jax
import jax.numpy as jnp
from jax.experimental import pallas as pl


def kernel(x, edge_index):
    raise NotImplementedError("write your pallas kernel here")


# speedup vs baseline: 5.5446x; 5.5446x over previous
import functools

import jax
import jax.numpy as jnp
from jax import lax
from jax.experimental import pallas as pl
from jax.experimental.pallas import tpu as pltpu
from jax.experimental.pallas import tpu_sc as plsc

N_NODES = 10000
N_EDGES = 320000
D_FEAT = 128

NUM_CORES = 2
NUM_SUBCORES = 16
NUM_WORKERS = NUM_CORES * NUM_SUBCORES
EDGES_PER_WORKER = N_EDGES // NUM_WORKERS
CHUNK = 80
NUM_CHUNKS = EDGES_PER_WORKER // CHUNK

ROWS_PER_TILE = 624
TAIL_START = ROWS_PER_TILE * NUM_SUBCORES
TAIL_ROWS = N_NODES - TAIL_START
ZERO_ROWS = 16


def _sc_partial_sums(x, src, dst):
    mesh = plsc.VectorSubcoreMesh(core_axis_name="c", subcore_axis_name="s")

    @functools.partial(
        pl.kernel,
        mesh=mesh,
        out_type=jax.ShapeDtypeStruct((NUM_CORES, N_NODES, D_FEAT), jnp.float32),
        scratch_types=[
            pltpu.VMEM((CHUNK,), jnp.int32),
            pltpu.VMEM((CHUNK,), jnp.int32),
            pltpu.VMEM((CHUNK, D_FEAT), jnp.float32),
            pltpu.VMEM((ZERO_ROWS, D_FEAT), jnp.float32),
            pltpu.VMEM_SHARED((N_NODES, D_FEAT), jnp.float32),
            pltpu.SemaphoreType.DMA,
        ],
    )
    def k(x_hbm, src_hbm, dst_hbm, out_hbm, src_v, dst_v, rows_v, zero_v, acc_sh, sem):
        cid = lax.axis_index("c")
        sid = lax.axis_index("s")
        wid = cid * NUM_SUBCORES + sid

        zvec = jnp.zeros((16,), jnp.float32)
        for i in range(ZERO_ROWS):
            for j in range(D_FEAT // 16):
                zero_v[i, pl.ds(j * 16, 16)] = zvec
        row0 = sid * ROWS_PER_TILE
        for i in range(ROWS_PER_TILE // ZERO_ROWS):
            pltpu.sync_copy(
                zero_v, acc_sh.at[pl.ds(row0 + i * ZERO_ROWS, ZERO_ROWS)]
            )

        @pl.when(sid == 0)
        def _zero_tail():
            pltpu.sync_copy(zero_v, acc_sh.at[pl.ds(TAIL_START, TAIL_ROWS)])

        plsc.subcore_barrier()

        def body(ch, carry):
            base = wid * EDGES_PER_WORKER + ch * CHUNK
            pltpu.sync_copy(src_hbm.at[pl.ds(base, CHUNK)], src_v)
            pltpu.sync_copy(dst_hbm.at[pl.ds(base, CHUNK)], dst_v)
            pltpu.async_copy(x_hbm.at[src_v], rows_v, sem).wait()
            pltpu.sync_copy(rows_v, acc_sh.at[dst_v], add=True)
            return carry

        lax.fori_loop(0, NUM_CHUNKS, body, 0)
        plsc.subcore_barrier()

        pltpu.sync_copy(
            acc_sh.at[pl.ds(row0, ROWS_PER_TILE)],
            out_hbm.at[cid, pl.ds(row0, ROWS_PER_TILE)],
        )

        @pl.when(sid == 0)
        def _write_tail():
            pltpu.sync_copy(
                acc_sh.at[pl.ds(TAIL_START, TAIL_ROWS)],
                out_hbm.at[cid, pl.ds(TAIL_START, TAIL_ROWS)],
            )

    return k(x, src, dst)


def _tc_add(partials):
    grid = 10
    rows = N_NODES // grid

    def add_kernel(a_ref, o_ref):
        o_ref[...] = a_ref[0] + a_ref[1]

    return pl.pallas_call(
        add_kernel,
        out_shape=jax.ShapeDtypeStruct((N_NODES, D_FEAT), jnp.float32),
        grid=(grid,),
        in_specs=[
            pl.BlockSpec((NUM_CORES, rows, D_FEAT), lambda i: (0, i, 0))
        ],
        out_specs=pl.BlockSpec((rows, D_FEAT), lambda i: (i, 0)),
    )(partials)


def kernel(x, edge_index):
    partials = _sc_partial_sums(x, edge_index[0], edge_index[1])
    return _tc_add(partials)


# speedup vs baseline: 13.9624x; 2.5182x over previous
import functools

import jax
import jax.numpy as jnp
from jax import lax
from jax.experimental import pallas as pl
from jax.experimental.pallas import tpu as pltpu
from jax.experimental.pallas import tpu_sc as plsc

N_NODES = 10000
N_EDGES = 320000
D_FEAT = 128

NUM_CORES = 2
NUM_SUBCORES = 16
NUM_WORKERS = NUM_CORES * NUM_SUBCORES
EDGES_PER_WORKER = N_EDGES // NUM_WORKERS
CHUNK = 80
NUM_CHUNKS = EDGES_PER_WORKER // CHUNK
NBUF = 4
NUM_MAIN = (NUM_CHUNKS - 1) // NBUF * NBUF
assert NUM_CHUNKS - NUM_MAIN == 1

ROWS_PER_TILE = 624
TAIL_START = ROWS_PER_TILE * NUM_SUBCORES
TAIL_ROWS = N_NODES - TAIL_START
ZERO_ROWS = 16


def _sc_partial_sums(x, src, dst):
    mesh = plsc.VectorSubcoreMesh(core_axis_name="c", subcore_axis_name="s")

    scratch = (
        [pltpu.VMEM((CHUNK,), jnp.int32) for _ in range(NBUF)]
        + [pltpu.VMEM((CHUNK,), jnp.int32) for _ in range(NBUF)]
        + [pltpu.VMEM((CHUNK, D_FEAT), jnp.float32) for _ in range(NBUF)]
        + [pltpu.VMEM((ZERO_ROWS, D_FEAT), jnp.float32)]
        + [pltpu.VMEM_SHARED((N_NODES, D_FEAT), jnp.float32)]
        + [pltpu.SemaphoreType.DMA] * (3 * NBUF)
    )

    @functools.partial(
        pl.kernel,
        mesh=mesh,
        out_type=jax.ShapeDtypeStruct((NUM_CORES, N_NODES, D_FEAT), jnp.float32),
        scratch_types=scratch,
    )
    def k(x_hbm, src_hbm, dst_hbm, out_hbm, *refs):
        srcb = refs[0:NBUF]
        dstb = refs[NBUF : 2 * NBUF]
        rowsb = refs[2 * NBUF : 3 * NBUF]
        zero_v = refs[3 * NBUF]
        acc_sh = refs[3 * NBUF + 1]
        sem_i = refs[3 * NBUF + 2 : 3 * NBUF + 2 + NBUF]
        sem_g = refs[3 * NBUF + 2 + NBUF : 3 * NBUF + 2 + 2 * NBUF]
        sem_s = refs[3 * NBUF + 2 + 2 * NBUF : 3 * NBUF + 2 + 3 * NBUF]

        cid = lax.axis_index("c")
        sid = lax.axis_index("s")
        wid = cid * NUM_SUBCORES + sid
        ebase = wid * EDGES_PER_WORKER

        zvec = jnp.zeros((16,), jnp.float32)
        for i in range(ZERO_ROWS):
            for j in range(D_FEAT // 16):
                zero_v[i, pl.ds(j * 16, 16)] = zvec
        row0 = sid * ROWS_PER_TILE
        for i in range(ROWS_PER_TILE // ZERO_ROWS):
            pltpu.sync_copy(
                zero_v, acc_sh.at[pl.ds(row0 + i * ZERO_ROWS, ZERO_ROWS)]
            )

        @pl.when(sid == 0)
        def _zero_tail():
            pltpu.sync_copy(zero_v, acc_sh.at[pl.ds(TAIL_START, TAIL_ROWS)])

        plsc.subcore_barrier()

        def issue_idx(c, sl):
            pltpu.async_copy(
                src_hbm.at[pl.ds(ebase + c * CHUNK, CHUNK)], srcb[sl], sem_i[sl]
            )
            pltpu.async_copy(
                dst_hbm.at[pl.ds(ebase + c * CHUNK, CHUNK)], dstb[sl], sem_i[sl]
            )

        def wait_idx(c, sl):
            pltpu.make_async_copy(
                src_hbm.at[pl.ds(ebase + c * CHUNK, CHUNK)], srcb[sl], sem_i[sl]
            ).wait()
            pltpu.make_async_copy(
                dst_hbm.at[pl.ds(ebase + c * CHUNK, CHUNK)], dstb[sl], sem_i[sl]
            ).wait()

        def issue_gather(sl):
            pltpu.async_copy(x_hbm.at[srcb[sl]], rowsb[sl], sem_g[sl])

        def wait_gather(sl):
            pltpu.make_async_copy(x_hbm.at[srcb[sl]], rowsb[sl], sem_g[sl]).wait()

        def issue_scatter(sl):
            pltpu.async_copy(rowsb[sl], acc_sh.at[dstb[sl]], sem_s[sl], add=True)

        def wait_scatter(sl):
            pltpu.make_async_copy(rowsb[sl], acc_sh.at[dstb[sl]], sem_s[sl]).wait()

        def body(g, carry):
            for b in range(NBUF):
                i = g + b
                sl = b
                sl1 = (b - 1) % NBUF
                sl2 = (b - 2) % NBUF

                @pl.when(i >= NBUF)
                def _drain():
                    wait_scatter(sl)

                issue_idx(i, sl)

                @pl.when(i >= 1)
                def _gather():
                    wait_idx(i - 1, sl1)
                    issue_gather(sl1)

                @pl.when(i >= 2)
                def _scatter():
                    wait_gather(sl2)
                    issue_scatter(sl2)

            return carry

        lax.fori_loop(0, NUM_MAIN // NBUF, lambda g, c: body(g * NBUF, c), 0)

        last = NUM_CHUNKS - 1
        wait_scatter(0)
        issue_idx(last, 0)
        wait_idx(last - 1, 3)
        issue_gather(3)
        wait_gather(2)
        issue_scatter(2)
        wait_idx(last, 0)
        issue_gather(0)
        wait_gather(3)
        issue_scatter(3)
        wait_gather(0)
        issue_scatter(0)
        for b in (1, 2, 3, 0):
            wait_scatter(b)

        plsc.subcore_barrier()

        pltpu.sync_copy(
            acc_sh.at[pl.ds(row0, ROWS_PER_TILE)],
            out_hbm.at[cid, pl.ds(row0, ROWS_PER_TILE)],
        )

        @pl.when(sid == 0)
        def _write_tail():
            pltpu.sync_copy(
                acc_sh.at[pl.ds(TAIL_START, TAIL_ROWS)],
                out_hbm.at[cid, pl.ds(TAIL_START, TAIL_ROWS)],
            )

    return k(x, src, dst)


def _tc_add(partials):
    grid = 10
    rows = N_NODES // grid

    def add_kernel(a_ref, o_ref):
        o_ref[...] = a_ref[0] + a_ref[1]

    return pl.pallas_call(
        add_kernel,
        out_shape=jax.ShapeDtypeStruct((N_NODES, D_FEAT), jnp.float32),
        grid=(grid,),
        in_specs=[
            pl.BlockSpec((NUM_CORES, rows, D_FEAT), lambda i: (0, i, 0))
        ],
        out_specs=pl.BlockSpec((rows, D_FEAT), lambda i: (i, 0)),
    )(partials)


def kernel(x, edge_index):
    partials = _sc_partial_sums(x, edge_index[0], edge_index[1])
    return _tc_add(partials)
